# baseline (speedup 1.0000x reference)
import functools

import jax
import jax.numpy as jnp
from jax import lax
from jax.experimental import pallas as pl
from jax.experimental.pallas import tpu as pltpu
from jax.experimental.pallas import tpu_sc as plsc

B = 16384
BC = 1024
D = 64
NC = 2
NS = 16
NW = NC * NS
BW = B // NW
BCW = BC // NW
S = 128
NSUB = BW // S


def _body(margin_h, aBHE_h, aBTE_h, aBC_h, nABHE_h, nABTE_h, nABC_h,
          uniqE_h, uniqBC_h, ent_h, bch_h, bct_h,
          memberL_h, discL_h, normL_h, alignL_h, countL_h,
          i_ahe, i_ate, i_ac, i_nhe, i_nte, i_nc, i_ue, i_bc,
          r_ahe, r_ate, r_ach, r_act, r_nhe, r_nte, r_nch, r_nct, r_ue,
          r_bh, r_bt,
          o_memb, o_disc, o_norm, o_align, o_count,
          t_a, t_b, t_c, margin_v, sem):
    cid = lax.axis_index("c")
    sid = lax.axis_index("s")
    wid = sid * NC + cid
    base = wid * BW

    pltpu.sync_copy(margin_h, margin_v)
    mvec = margin_v[...]
    iot = lax.iota(jnp.int32, 16)

    def transpose_sum(t_ref):
        acc = plsc.load_gather(t_ref, [iot, jnp.zeros((16,), jnp.int32)])
        for l in range(1, 16):
            acc = acc + plsc.load_gather(
                t_ref, [iot, jnp.full((16,), l, jnp.int32)])
        return acc

    for sub in range(NSUB):
        off = base + sub * S
        pltpu.sync_copy(aBHE_h.at[pl.ds(off, S)], i_ahe)
        pltpu.sync_copy(aBTE_h.at[pl.ds(off, S)], i_ate)
        pltpu.sync_copy(aBC_h.at[pl.ds(off, S)], i_ac)
        pltpu.sync_copy(nABHE_h.at[pl.ds(off, S)], i_nhe)
        pltpu.sync_copy(nABTE_h.at[pl.ds(off, S)], i_nte)
        pltpu.sync_copy(nABC_h.at[pl.ds(off, S)], i_nc)
        pltpu.sync_copy(uniqE_h.at[pl.ds(off, S)], i_ue)

        cps = [
            pltpu.async_copy(ent_h.at[i_ahe], r_ahe, sem),
            pltpu.async_copy(ent_h.at[i_ate], r_ate, sem),
            pltpu.async_copy(bch_h.at[i_ac], r_ach, sem),
            pltpu.async_copy(bct_h.at[i_ac], r_act, sem),
            pltpu.async_copy(ent_h.at[i_nhe], r_nhe, sem),
            pltpu.async_copy(ent_h.at[i_nte], r_nte, sem),
            pltpu.async_copy(bch_h.at[i_nc], r_nch, sem),
            pltpu.async_copy(bct_h.at[i_nc], r_nct, sem),
            pltpu.async_copy(ent_h.at[i_ue], r_ue, sem),
        ]
        for cp in cps:
            cp.wait()

        def group(g, carry, sub=sub):
            def item(j, c2):
                i = g * 16 + j
                macc = jnp.zeros((16,), jnp.float32)
                nacc = jnp.zeros((16,), jnp.float32)
                uacc = jnp.zeros((16,), jnp.float32)
                for c4 in range(D // 16):
                    sl = pl.ds(c4 * 16, 16)
                    th = (1.0 - r_ach[i, sl]) * r_ahe[i, sl]
                    tt = (1.0 - r_act[i, sl]) * r_ate[i, sl]
                    macc = macc + th * th + tt * tt
                    nh = (1.0 - r_nch[i, sl]) * r_nhe[i, sl]
                    nt = (1.0 - r_nct[i, sl]) * r_nte[i, sl]
                    nacc = nacc + nh * nh + nt * nt
                    ue = r_ue[i, sl]
                    uacc = uacc + ue * ue
                t_a[j, :] = macc
                t_b[j, :] = nacc
                t_c[j, :] = uacc
                return c2

            lax.fori_loop(0, 16, item, 0)
            ob = sub * S + g * 16
            o_memb[pl.ds(ob, 16)] = transpose_sum(t_a)
            o_disc[pl.ds(ob, 16)] = jnp.maximum(mvec - transpose_sum(t_b), 0.0)
            us = transpose_sum(t_c) - 1.0
            o_norm[pl.ds(ob, 16)] = us * us
            return carry

        lax.fori_loop(0, S // 16, group, 0)

    pltpu.sync_copy(o_memb, memberL_h.at[pl.ds(base, BW)])
    pltpu.sync_copy(o_disc, discL_h.at[pl.ds(base, BW)])
    pltpu.sync_copy(o_norm, normL_h.at[pl.ds(base, BW)])

    bco = wid * BCW
    pltpu.sync_copy(uniqBC_h.at[pl.ds(bco, BCW)], i_bc)
    cpb = [
        pltpu.async_copy(bch_h.at[i_bc], r_bh, sem),
        pltpu.async_copy(bct_h.at[i_bc], r_bt, sem),
    ]
    for cp in cpb:
        cp.wait()

    def bgroup(g, carry):
        def bitem(j, c2):
            i = g * 16 + j
            aacc = jnp.zeros((16,), jnp.float32)
            habs = jnp.zeros((16,), jnp.float32)
            tabs = jnp.zeros((16,), jnp.float32)
            for c4 in range(D // 16):
                sl = pl.ds(c4 * 16, 16)
                h = r_bh[i, sl]
                t = r_bt[i, sl]
                ph = h * (1.0 - h)
                ptt = t * (1.0 - t)
                aacc = aacc + ph * ph + ptt * ptt
                habs = habs + jnp.abs(h)
                tabs = tabs + jnp.abs(t)
            t_a[j, :] = aacc
            t_b[j, :] = habs
            t_c[j, :] = tabs
            return c2

        lax.fori_loop(0, 16, bitem, 0)
        ob = g * 16
        o_align[pl.ds(ob, 16)] = transpose_sum(t_a)
        o_count[pl.ds(ob, 16)] = (
            jnp.maximum(1.0 - transpose_sum(t_b), 0.0)
            + jnp.maximum(1.0 - transpose_sum(t_c), 0.0))
        return carry

    lax.fori_loop(0, BCW // 16, bgroup, 0)

    pltpu.sync_copy(o_align, alignL_h.at[pl.ds(bco, BCW)])
    pltpu.sync_copy(o_count, countL_h.at[pl.ds(bco, BCW)])


@jax.jit
def _run(margin, aBHE, aBTE, aBC, nABHE, nABTE, nABC, uniqE, uniqBC,
         entityEmbed, bConceptHEmbed, bConceptTEmbed):
    mesh = plsc.VectorSubcoreMesh(core_axis_name="c", subcore_axis_name="s")
    f32 = jnp.float32
    i32 = jnp.int32
    kern = functools.partial(
        pl.kernel,
        out_type=[
            jax.ShapeDtypeStruct((B,), f32),
            jax.ShapeDtypeStruct((B,), f32),
            jax.ShapeDtypeStruct((B,), f32),
            jax.ShapeDtypeStruct((BC,), f32),
            jax.ShapeDtypeStruct((BC,), f32),
        ],
        mesh=mesh,
        compiler_params=pltpu.CompilerParams(
            needs_layout_passes=False, use_tc_tiling_on_sc=False),
        scratch_types=[
            pltpu.VMEM((S,), i32), pltpu.VMEM((S,), i32),
            pltpu.VMEM((S,), i32), pltpu.VMEM((S,), i32),
            pltpu.VMEM((S,), i32), pltpu.VMEM((S,), i32),
            pltpu.VMEM((S,), i32), pltpu.VMEM((BCW,), i32),
            pltpu.VMEM((S, D), f32), pltpu.VMEM((S, D), f32),
            pltpu.VMEM((S, D), f32), pltpu.VMEM((S, D), f32),
            pltpu.VMEM((S, D), f32), pltpu.VMEM((S, D), f32),
            pltpu.VMEM((S, D), f32), pltpu.VMEM((S, D), f32),
            pltpu.VMEM((S, D), f32),
            pltpu.VMEM((BCW, D), f32), pltpu.VMEM((BCW, D), f32),
            pltpu.VMEM((BW,), f32), pltpu.VMEM((BW,), f32),
            pltpu.VMEM((BW,), f32),
            pltpu.VMEM((BCW,), f32), pltpu.VMEM((BCW,), f32),
            pltpu.VMEM((16, 16), f32), pltpu.VMEM((16, 16), f32),
            pltpu.VMEM((16, 16), f32),
            pltpu.VMEM((16,), f32),
            pltpu.SemaphoreType.DMA,
        ],
    )(_body)
    return kern(margin, aBHE, aBTE, aBC, nABHE, nABTE, nABC, uniqE, uniqBC,
                entityEmbed, bConceptHEmbed, bConceptTEmbed)


def kernel(aBHE, aBTE, aBC, nABHE, nABTE, nABC, uniqE, uniqBC, lossMargin,
           device, entityEmbed, bConceptHEmbed, bConceptTEmbed):
    del device
    margin = jnp.broadcast_to(jnp.asarray(lossMargin, jnp.float32), (16,))
    cast = lambda x: x.astype(jnp.int32)
    out = _run(margin, cast(aBHE), cast(aBTE), cast(aBC), cast(nABHE),
               cast(nABTE), cast(nABC), cast(uniqE), cast(uniqBC),
               entityEmbed, bConceptHEmbed, bConceptTEmbed)
    return tuple(out)
